# Initial kernel scaffold; baseline (speedup 1.0000x reference)
import jax
import jax.numpy as jnp
from jax.experimental import pallas as pl


def kernel(z):
    raise NotImplementedError("write your pallas kernel here")


# speedup vs baseline: 12.4359x; 12.4359x over previous
import functools

import jax
import jax.numpy as jnp
from jax import lax
from jax.experimental import pallas as pl
from jax.experimental.pallas import tpu as pltpu
from jax.experimental.pallas import tpu_sc as plsc

ROWS = 128
COLS = 32768
TOPK = int(0.1 * COLS)
L = 16
NC, NS = 2, 16
NW = NC * NS
ROWS_PER_W = ROWS // NW
SHIFT = 18
NBUCK = 1 << (32 - SHIFT)
BIAS = NBUCK // 2

_MASK31 = 0x7FFFFFFF
_INT_MIN = -(2**31)


def _key16(v):
    i = lax.bitcast_convert_type(v, jnp.int32)
    return i ^ ((i >> 31) & _MASK31)


def _make_sc_kernel():
    mesh = plsc.VectorSubcoreMesh(
        core_axis_name="c", subcore_axis_name="s", num_cores=NC, num_subcores=NS
    )

    @functools.partial(
        pl.kernel,
        out_type=jax.ShapeDtypeStruct((ROWS, COLS), jnp.float32),
        mesh=mesh,
        compiler_params=pltpu.CompilerParams(needs_layout_passes=False),
        scratch_types=[
            pltpu.VMEM((COLS,), jnp.float32),
            pltpu.VMEM((NBUCK,), jnp.int32),
            pltpu.VMEM((COLS + L,), jnp.int32),
        ],
    )
    def sc_topk_mask(z_hbm, out_hbm, row_v, hist_v, cand_v):
        cid = lax.axis_index("c")
        sid = lax.axis_index("s")
        wid = sid * NC + cid

        iota = lax.iota(jnp.int32, L)
        zeros16 = jnp.zeros((L,), jnp.int32)
        ones16 = jnp.ones((L,), jnp.int32)
        zeros16f = jnp.zeros((L,), jnp.float32)

        def row_body(j, _):
            row = wid * ROWS_PER_W + j
            pltpu.sync_copy(z_hbm.at[row], row_v)

            def clr(i, c):
                hist_v[pl.ds(i * L, L)] = zeros16
                return c

            lax.fori_loop(0, NBUCK // L, clr, 0)

            def p1(i, mx):
                key = _key16(row_v[pl.ds(i * L, L)])
                b = (key >> SHIFT) + BIAS
                plsc.addupdate_scatter(hist_v, [b], ones16)
                return jnp.maximum(mx, key)

            mxv = lax.fori_loop(
                0, COLS // L, p1, jnp.full((L,), _INT_MIN, jnp.int32)
            )
            kmax = jnp.max(mxv)

            vb0 = ((kmax >> SHIFT) + BIAS) // L

            def scond(c):
                vb, acc = c
                s = jnp.sum(hist_v[pl.ds(vb * L, L)])
                return acc + s < TOPK

            def sbody(c):
                vb, acc = c
                s = jnp.sum(hist_v[pl.ds(vb * L, L)])
                return vb - 1, acc + s

            vb, acc = lax.while_loop(scond, sbody, (vb0, jnp.int32(0)))

            h = hist_v[pl.ds(vb * L, L)]
            s = jnp.sum(h)
            cs = plsc.cumsum(h)
            incl = jnp.max(cs) == s
            suffix = s - cs + jnp.where(incl, h, zeros16)
            cond = (acc + suffix) >= TOPK
            jB = jnp.max(jnp.where(cond, iota, jnp.int32(-1)))
            q = TOPK - acc - suffix + h
            kprime = jnp.max(jnp.where(iota == jB, q, _INT_MIN))
            B = vb * L + jB

            def p2(i, off):
                key = _key16(row_v[pl.ds(i * L, L)])
                m = ((key >> SHIFT) + BIAS) == B
                plsc.store_compressed(cand_v.at[pl.ds(off, L)], key, mask=m)
                return off + jnp.max(plsc.all_reduce_population_count(m))

            n = lax.fori_loop(0, COLS // L, p2, jnp.int32(0))
            nv = (n + L - 1) // L

            prefix = lax.shift_left(B - BIAS, SHIFT)

            def bs(bi, t):
                trial = t | lax.shift_left(jnp.int32(1), SHIFT - 1 - bi)

                def cnt(ci, accv):
                    ck = cand_v[pl.ds(ci * L, L)]
                    valid = (ci * L + iota) < n
                    return accv + jnp.where(valid & (ck >= trial), 1, 0)

                c = jnp.sum(lax.fori_loop(0, nv, cnt, zeros16))
                return jnp.where(c >= kprime, trial, t)

            t = lax.fori_loop(0, SHIFT, bs, prefix)

            tvec = jnp.full((L,), t, jnp.int32)
            tf = lax.bitcast_convert_type(tvec ^ ((tvec >> 31) & _MASK31), jnp.float32)

            def p3(i, c):
                v = row_v[pl.ds(i * L, L)]
                row_v[pl.ds(i * L, L)] = jnp.where(v >= tf, v, zeros16f)
                return c

            lax.fori_loop(0, COLS // L, p3, 0)
            pltpu.sync_copy(row_v, out_hbm.at[row])
            return 0

        lax.fori_loop(0, ROWS_PER_W, row_body, 0)

    return sc_topk_mask


_sc_kernel = _make_sc_kernel()


@jax.jit
def kernel(z):
    return _sc_kernel(z)


# speedup vs baseline: 14.0124x; 1.1268x over previous
import functools

import jax
import jax.numpy as jnp
from jax import lax
from jax.experimental import pallas as pl
from jax.experimental.pallas import tpu as pltpu
from jax.experimental.pallas import tpu_sc as plsc

ROWS = 128
COLS = 32768
TOPK = int(0.1 * COLS)
L = 16
NC, NS = 2, 16
NW = NC * NS
ROWS_PER_W = ROWS // NW
SHIFT = 18
NBUCK = 1 << (32 - SHIFT)
BIAS = NBUCK // 2
UNROLL = 8

_MASK31 = 0x7FFFFFFF
_INT_MIN = -(2**31)


def _key16(v):
    i = lax.bitcast_convert_type(v, jnp.int32)
    return i ^ ((i >> 31) & _MASK31)


def _make_sc_kernel():
    mesh = plsc.VectorSubcoreMesh(
        core_axis_name="c", subcore_axis_name="s", num_cores=NC, num_subcores=NS
    )

    @functools.partial(
        pl.kernel,
        out_type=jax.ShapeDtypeStruct((ROWS, COLS), jnp.float32),
        mesh=mesh,
        compiler_params=pltpu.CompilerParams(needs_layout_passes=False),
        scratch_types=[
            pltpu.VMEM((COLS,), jnp.float32),
            pltpu.VMEM((COLS,), jnp.float32),
            pltpu.VMEM((NBUCK,), jnp.int32),
            pltpu.VMEM((COLS + L,), jnp.int32),
            pltpu.SemaphoreType.DMA,
            pltpu.SemaphoreType.DMA,
            pltpu.SemaphoreType.DMA,
            pltpu.SemaphoreType.DMA,
        ],
    )
    def sc_topk_mask(z_hbm, out_hbm, row_a, row_b, hist_v, cand_v,
                     sin_a, sin_b, sout_a, sout_b):
        cid = lax.axis_index("c")
        sid = lax.axis_index("s")
        wid = sid * NC + cid
        row0 = wid * ROWS_PER_W

        iota = lax.iota(jnp.int32, L)
        zeros16 = jnp.zeros((L,), jnp.int32)
        ones16 = jnp.ones((L,), jnp.int32)
        zeros16f = jnp.zeros((L,), jnp.float32)
        incl_cs = jnp.max(plsc.cumsum(ones16)) == L

        def process(buf):
            def clr(i, c):
                for u in range(UNROLL):
                    hist_v[pl.ds((i * UNROLL + u) * L, L)] = zeros16
                return c

            lax.fori_loop(0, NBUCK // L // UNROLL, clr, 0)

            def p1(i, mx):
                for u in range(UNROLL):
                    key = _key16(buf[pl.ds((i * UNROLL + u) * L, L)])
                    b = (key >> SHIFT) + BIAS
                    plsc.addupdate_scatter(hist_v, [b], ones16)
                    mx = jnp.maximum(mx, key)
                return mx

            mxv = lax.fori_loop(
                0, COLS // L // UNROLL, p1, jnp.full((L,), _INT_MIN, jnp.int32)
            )
            kmax = jnp.max(mxv)

            vb0 = ((kmax >> SHIFT) + BIAS) // L

            def scond(c):
                vb, acc = c
                s = jnp.sum(hist_v[pl.ds(vb * L, L)])
                return acc + s < TOPK

            def sbody(c):
                vb, acc = c
                s = jnp.sum(hist_v[pl.ds(vb * L, L)])
                return vb - 1, acc + s

            vb, acc = lax.while_loop(scond, sbody, (vb0, jnp.int32(0)))

            h = hist_v[pl.ds(vb * L, L)]
            s = jnp.sum(h)
            cs = plsc.cumsum(h)
            suffix = s - cs + jnp.where(incl_cs, h, zeros16)
            cond = (acc + suffix) >= TOPK
            jB = jnp.max(jnp.where(cond, iota, jnp.int32(-1)))
            q = TOPK - acc - suffix + h
            kprime = jnp.max(jnp.where(iota == jB, q, _INT_MIN))
            B = vb * L + jB

            def p2(i, off_v):
                for u in range(UNROLL):
                    key = _key16(buf[pl.ds((i * UNROLL + u) * L, L)])
                    m = ((key >> SHIFT) + BIAS) == B
                    m01 = jnp.where(m, 1, 0)
                    cs2 = plsc.cumsum(m01)
                    excl = cs2 - jnp.where(incl_cs, m01, zeros16)
                    plsc.store_scatter(cand_v, [off_v + excl], key, mask=m)
                    off_v = off_v + plsc.all_reduce_population_count(m)
                return off_v

            off_v = lax.fori_loop(0, COLS // L // UNROLL, p2, zeros16)
            n = jnp.max(off_v)
            nv = (n + L - 1) // L

            prefix = lax.shift_left(B - BIAS, SHIFT)

            def bs(bi, t):
                trial = t | lax.shift_left(jnp.int32(1), SHIFT - 1 - bi)

                def cnt(ci, accv):
                    ck = cand_v[pl.ds(ci * L, L)]
                    valid = (ci * L + iota) < n
                    return accv + jnp.where(valid & (ck >= trial), 1, 0)

                c = jnp.sum(lax.fori_loop(0, nv, cnt, zeros16))
                return jnp.where(c >= kprime, trial, t)

            t = lax.fori_loop(0, SHIFT, bs, prefix)

            tvec = jnp.full((L,), t, jnp.int32)
            tf = lax.bitcast_convert_type(tvec ^ ((tvec >> 31) & _MASK31), jnp.float32)

            def p3(i, c):
                for u in range(UNROLL):
                    sl = pl.ds((i * UNROLL + u) * L, L)
                    v = buf[sl]
                    buf[sl] = jnp.where(v >= tf, v, zeros16f)
                return c

            lax.fori_loop(0, COLS // L // UNROLL, p3, 0)

        bufs = [row_a, row_b]
        sins = [sin_a, sin_b]
        souts = [sout_a, sout_b]
        cp_in = pltpu.async_copy(z_hbm.at[row0], row_a, sin_a)
        cp_outs = [None] * ROWS_PER_W
        for j in range(ROWS_PER_W):
            buf = bufs[j % 2]
            if j >= 1:
                cp_outs[j - 1].wait()
            if j + 1 < ROWS_PER_W:
                cp_next = pltpu.async_copy(
                    z_hbm.at[row0 + j + 1], bufs[(j + 1) % 2], sins[(j + 1) % 2]
                )
            cp_in.wait()
            process(buf)
            cp_outs[j] = pltpu.async_copy(buf, out_hbm.at[row0 + j], souts[j % 2])
            if j + 1 < ROWS_PER_W:
                cp_in = cp_next
        cp_outs[ROWS_PER_W - 1].wait()

    return sc_topk_mask


_sc_kernel = _make_sc_kernel()


@jax.jit
def kernel(z):
    return _sc_kernel(z)


# speedup vs baseline: 42.1653x; 3.0091x over previous
import functools

import jax
import jax.numpy as jnp
from jax import lax
from jax.experimental import pallas as pl
from jax.experimental.pallas import tpu as pltpu
from jax.experimental.pallas import tpu_sc as plsc

ROWS = 128
COLS = 32768
TOPK = int(0.1 * COLS)
L = 16
NC, NS = 2, 16
NW = NC * NS
ROWS_PER_W = ROWS // NW
SHIFT = 18
NBUCK = 1 << (32 - SHIFT)
BIAS = NBUCK // 2
UNROLL = 8

_MASK31 = 0x7FFFFFFF
_INT_MIN = -(2**31)


def _key16(v):
    i = lax.bitcast_convert_type(v, jnp.int32)
    return i ^ ((i >> 31) & _MASK31)


def _make_sc_kernel():
    mesh = plsc.VectorSubcoreMesh(
        core_axis_name="c", subcore_axis_name="s", num_cores=NC, num_subcores=NS
    )

    @functools.partial(
        pl.kernel,
        out_type=jax.ShapeDtypeStruct((ROWS, COLS), jnp.float32),
        mesh=mesh,
        compiler_params=pltpu.CompilerParams(needs_layout_passes=False),
        scratch_types=[
            pltpu.VMEM((COLS,), jnp.float32),
            pltpu.VMEM((COLS,), jnp.float32),
            pltpu.VMEM((NBUCK,), jnp.int32),
            pltpu.VMEM((COLS + L,), jnp.int32),
            pltpu.SemaphoreType.DMA,
            pltpu.SemaphoreType.DMA,
            pltpu.SemaphoreType.DMA,
            pltpu.SemaphoreType.DMA,
        ],
    )
    def sc_topk_mask(z_hbm, out_hbm, row_a, row_b, hist_v, cand_v,
                     sin_a, sin_b, sout_a, sout_b):
        cid = lax.axis_index("c")
        sid = lax.axis_index("s")
        wid = sid * NC + cid
        row0 = wid * ROWS_PER_W

        iota = lax.iota(jnp.int32, L)
        zeros16 = jnp.zeros((L,), jnp.int32)
        ones16 = jnp.ones((L,), jnp.int32)
        zeros16f = jnp.zeros((L,), jnp.float32)
        incl_cs = jnp.max(plsc.cumsum(ones16)) == L

        def process(buf):
            @plsc.parallel_loop(0, NBUCK // L, unroll=UNROLL)
            def _clr(i):
                hist_v[pl.ds(i * L, L)] = zeros16

            @plsc.parallel_loop(0, COLS // L, unroll=UNROLL,
                                carry=jnp.full((L,), _INT_MIN, jnp.int32))
            def mxv(i, mx):
                key = _key16(buf[pl.ds(i * L, L)])
                b = (key >> SHIFT) + BIAS
                plsc.addupdate_scatter(hist_v, [b], ones16)
                return jnp.maximum(mx, key)

            kmax = jnp.max(mxv)

            vb0 = ((kmax >> SHIFT) + BIAS) // L

            def scond(c):
                vb, acc = c
                s = jnp.sum(hist_v[pl.ds(vb * L, L)])
                return acc + s < TOPK

            def sbody(c):
                vb, acc = c
                s = jnp.sum(hist_v[pl.ds(vb * L, L)])
                return vb - 1, acc + s

            vb, acc = lax.while_loop(scond, sbody, (vb0, jnp.int32(0)))

            h = hist_v[pl.ds(vb * L, L)]
            s = jnp.sum(h)
            cs = plsc.cumsum(h)
            suffix = s - cs + jnp.where(incl_cs, h, zeros16)
            cond = (acc + suffix) >= TOPK
            jB = jnp.max(jnp.where(cond, iota, jnp.int32(-1)))
            q = TOPK - acc - suffix + h
            kprime = jnp.max(jnp.where(iota == jB, q, _INT_MIN))
            B = vb * L + jB

            @plsc.parallel_loop(0, COLS // L, unroll=UNROLL, carry=zeros16)
            def off_v(i, off):
                key = _key16(buf[pl.ds(i * L, L)])
                m = ((key >> SHIFT) + BIAS) == B
                m01 = jnp.where(m, 1, 0)
                cs2 = plsc.cumsum(m01)
                excl = cs2 - jnp.where(incl_cs, m01, zeros16)
                plsc.store_scatter(cand_v, [off + excl], key, mask=m)
                return off + plsc.all_reduce_population_count(m)

            n = jnp.max(off_v)
            nv = (n + L - 1) // L

            prefix = lax.shift_left(B - BIAS, SHIFT)

            def bs(bi, t):
                trial = t | lax.shift_left(jnp.int32(1), SHIFT - 1 - bi)

                def cnt(ci, accv):
                    ck = cand_v[pl.ds(ci * L, L)]
                    valid = (ci * L + iota) < n
                    return accv + jnp.where(valid & (ck >= trial), 1, 0)

                c = jnp.sum(lax.fori_loop(0, nv, cnt, zeros16))
                return jnp.where(c >= kprime, trial, t)

            t = lax.fori_loop(0, SHIFT, bs, prefix)

            tvec = jnp.full((L,), t, jnp.int32)
            tf = lax.bitcast_convert_type(tvec ^ ((tvec >> 31) & _MASK31), jnp.float32)

            @plsc.parallel_loop(0, COLS // L, unroll=UNROLL)
            def _p3(i):
                sl = pl.ds(i * L, L)
                v = buf[sl]
                buf[sl] = jnp.where(v >= tf, v, zeros16f)

        bufs = [row_a, row_b]
        sins = [sin_a, sin_b]
        souts = [sout_a, sout_b]
        cp_in = pltpu.async_copy(z_hbm.at[row0], row_a, sin_a)
        cp_outs = [None] * ROWS_PER_W
        for j in range(ROWS_PER_W):
            buf = bufs[j % 2]
            if j >= 1:
                cp_outs[j - 1].wait()
            if j + 1 < ROWS_PER_W:
                cp_next = pltpu.async_copy(
                    z_hbm.at[row0 + j + 1], bufs[(j + 1) % 2], sins[(j + 1) % 2]
                )
            cp_in.wait()
            process(buf)
            cp_outs[j] = pltpu.async_copy(buf, out_hbm.at[row0 + j], souts[j % 2])
            if j + 1 < ROWS_PER_W:
                cp_in = cp_next
        cp_outs[ROWS_PER_W - 1].wait()

    return sc_topk_mask


_sc_kernel = _make_sc_kernel()


@jax.jit
def kernel(z):
    return _sc_kernel(z)
